# baseline (speedup 1.0000x reference)
import jax
import jax.numpy as jnp
from jax.experimental import pallas as pl
from jax.experimental.pallas import tpu as pltpu


def _gcn_graph_kernel(xs_ref, as_ref, w_in_ref, b_in_ref, w0_ref, b0_ref,
                      w1_ref, b1_ref, w2_ref, b2_ref, y_ref):
    xs = xs_ref[0]
    adj = as_ref[0]

    def mm(a, b):
        return jnp.dot(a, b, preferred_element_type=jnp.float32)

    def bf(a):
        return a.astype(jnp.bfloat16)

    x0 = jax.nn.relu(mm(xs, w_in_ref[...]) + b_in_ref[...])
    x = jax.nn.relu(mm(bf(mm(adj, bf(x0))), w0_ref[...]) + b0_ref[...])
    x = jax.nn.relu(mm(bf(mm(adj, bf(x))), w1_ref[...]) + b1_ref[...])
    x = jnp.tanh(mm(bf(mm(adj, bf(x))), w2_ref[...]) + b2_ref[...])
    y_ref[0] = x + x0


def _mean_gather_kernel(idx_ref, y_ref, m_ref, out_ref):
    m = m_ref[0]
    y = y_ref[0]
    s = jnp.dot(m, y, preferred_element_type=jnp.float32)
    cnt = jnp.maximum(jnp.sum(m), 1.0)
    out_ref[0] = s / cnt


def kernel(cdfg_xs, cdfg_as, W_in, b_in, W0, b0, W1, b1, W2, b2, graph,
           coverpoint_mask):
    G, N, F = cdfg_xs.shape
    H = W_in.shape[1]
    B = graph.shape[0]

    bf = lambda a: a.astype(jnp.bfloat16)
    biases = [b.reshape(1, H) for b in (b_in, b0, b1, b2)]
    full = lambda *shape: pl.BlockSpec(shape, lambda g: (0,) * len(shape))

    y = pl.pallas_call(
        _gcn_graph_kernel,
        grid=(G,),
        in_specs=[
            pl.BlockSpec((1, N, F), lambda g: (g, 0, 0)),
            pl.BlockSpec((1, N, N), lambda g: (g, 0, 0)),
            full(F, H), full(1, H),
            full(H, H), full(1, H),
            full(H, H), full(1, H),
            full(H, H), full(1, H),
        ],
        out_specs=pl.BlockSpec((1, N, H), lambda g: (g, 0, 0)),
        out_shape=jax.ShapeDtypeStruct((G, N, H), jnp.float32),
        compiler_params=pltpu.CompilerParams(
            dimension_semantics=("parallel",)),
    )(bf(cdfg_xs), bf(cdfg_as), bf(W_in), biases[0], bf(W0), biases[1],
      bf(W1), biases[2], bf(W2), biases[3])

    idx = graph[:, 0].astype(jnp.int32)
    mask_f = coverpoint_mask.astype(jnp.float32).reshape(B, 1, N)

    out = pl.pallas_call(
        _mean_gather_kernel,
        grid_spec=pltpu.PrefetchScalarGridSpec(
            num_scalar_prefetch=1,
            grid=(B,),
            in_specs=[
                pl.BlockSpec((1, N, H), lambda b, idx_ref: (idx_ref[b], 0, 0)),
                pl.BlockSpec((1, 1, N), lambda b, idx_ref: (b, 0, 0)),
            ],
            out_specs=pl.BlockSpec((1, 1, H), lambda b, idx_ref: (b, 0, 0)),
        ),
        out_shape=jax.ShapeDtypeStruct((B, 1, H), jnp.float32),
    )(idx, y, mask_f)

    return out.reshape(B, H)


# speedup vs baseline: 1.4698x; 1.4698x over previous
import jax
import jax.numpy as jnp
from jax.experimental import pallas as pl
from jax.experimental.pallas import tpu as pltpu


def _gcn_graph_kernel(xs_ref, as_ref, w_in_ref, b_in_ref, w0_ref, b0_ref,
                      w1_ref, b1_ref, w2_ref, b2_ref, y_ref):
    def bf(a):
        return a.astype(jnp.bfloat16)

    def mm(a, b):
        return jnp.dot(a, b, preferred_element_type=jnp.float32)

    xs = bf(xs_ref[0])
    adj = bf(as_ref[0])
    w_in, w0, w1, w2 = (bf(w_in_ref[...]), bf(w0_ref[...]),
                        bf(w1_ref[...]), bf(w2_ref[...]))

    x0 = jax.nn.relu(mm(xs, w_in) + b_in_ref[...])
    x = jax.nn.relu(mm(bf(mm(adj, bf(x0))), w0) + b0_ref[...])
    x = jax.nn.relu(mm(bf(mm(adj, bf(x))), w1) + b1_ref[...])
    x = jnp.tanh(mm(bf(mm(adj, bf(x))), w2) + b2_ref[...])
    y_ref[0] = x + x0


def _mean_gather_kernel(idx_ref, y_ref, m_ref, out_ref):
    m = m_ref[0]
    y = y_ref[0]
    s = jnp.dot(m, y, preferred_element_type=jnp.float32)
    cnt = jnp.maximum(jnp.sum(m), 1.0)
    out_ref[0] = s / cnt


def kernel(cdfg_xs, cdfg_as, W_in, b_in, W0, b0, W1, b1, W2, b2, graph,
           coverpoint_mask):
    G, N, F = cdfg_xs.shape
    H = W_in.shape[1]
    B = graph.shape[0]

    biases = [b.reshape(1, H) for b in (b_in, b0, b1, b2)]
    full = lambda *shape: pl.BlockSpec(shape, lambda g: (0,) * len(shape))

    y = pl.pallas_call(
        _gcn_graph_kernel,
        grid=(G,),
        in_specs=[
            pl.BlockSpec((1, N, F), lambda g: (g, 0, 0)),
            pl.BlockSpec((1, N, N), lambda g: (g, 0, 0)),
            full(F, H), full(1, H),
            full(H, H), full(1, H),
            full(H, H), full(1, H),
            full(H, H), full(1, H),
        ],
        out_specs=pl.BlockSpec((1, N, H), lambda g: (g, 0, 0)),
        out_shape=jax.ShapeDtypeStruct((G, N, H), jnp.float32),
        compiler_params=pltpu.CompilerParams(
            dimension_semantics=("parallel",)),
    )(cdfg_xs, cdfg_as, W_in, biases[0], W0, biases[1],
      W1, biases[2], W2, biases[3])

    idx = graph[:, 0].astype(jnp.int32)
    mask_f = coverpoint_mask.astype(jnp.float32).reshape(B, 1, N)

    out = pl.pallas_call(
        _mean_gather_kernel,
        grid_spec=pltpu.PrefetchScalarGridSpec(
            num_scalar_prefetch=1,
            grid=(B,),
            in_specs=[
                pl.BlockSpec((1, N, H), lambda b, idx_ref: (idx_ref[b], 0, 0)),
                pl.BlockSpec((1, 1, N), lambda b, idx_ref: (b, 0, 0)),
            ],
            out_specs=pl.BlockSpec((1, 1, H), lambda b, idx_ref: (b, 0, 0)),
        ),
        out_shape=jax.ShapeDtypeStruct((B, 1, H), jnp.float32),
    )(idx, y, mask_f)

    return out.reshape(B, H)


# speedup vs baseline: 1.8597x; 1.2652x over previous
import jax
import jax.numpy as jnp
from jax.experimental import pallas as pl
from jax.experimental.pallas import tpu as pltpu


def _fused_kernel(xs_ref, as_ref, w_in_ref, b_in_ref, w0_ref, b0_ref,
                  w1_ref, b1_ref, w2_ref, b2_ref, idx_ref, m_ref, out_ref):
    g = pl.program_id(0)
    ng = pl.num_programs(0)

    def bf(a):
        return a.astype(jnp.bfloat16)

    def mm(a, b):
        return jnp.dot(a, b, preferred_element_type=jnp.float32)

    xs = bf(xs_ref[0])
    adj = bf(as_ref[0])

    x0 = jax.nn.relu(mm(xs, bf(w_in_ref[...])) + b_in_ref[...])
    x = jax.nn.relu(mm(bf(mm(adj, bf(x0))), bf(w0_ref[...])) + b0_ref[...])
    x = jax.nn.relu(mm(bf(mm(adj, bf(x))), bf(w1_ref[...])) + b1_ref[...])
    x = jnp.tanh(mm(bf(mm(adj, bf(x))), bf(w2_ref[...])) + b2_ref[...])
    y = x + x0

    mask = m_ref[...]
    sel = (idx_ref[...] == g).astype(jnp.float32)
    part = mm(mask * sel, y)

    prev = jnp.where(g == 0, 0.0, out_ref[...])
    acc = prev + part
    cnt = jnp.maximum(jnp.sum(mask, axis=1, keepdims=True), 1.0)
    out_ref[...] = jnp.where(g == ng - 1, acc / cnt, acc)


def kernel(cdfg_xs, cdfg_as, W_in, b_in, W0, b0, W1, b1, W2, b2, graph,
           coverpoint_mask):
    G, N, F = cdfg_xs.shape
    H = W_in.shape[1]
    B = graph.shape[0]

    biases = [b.reshape(1, H) for b in (b_in, b0, b1, b2)]
    idx = graph.reshape(B, 1).astype(jnp.int32)
    mask_f = coverpoint_mask.astype(jnp.float32)

    full = lambda *shape: pl.BlockSpec(shape, lambda g: (0,) * len(shape))

    out = pl.pallas_call(
        _fused_kernel,
        grid=(G,),
        in_specs=[
            pl.BlockSpec((1, N, F), lambda g: (g, 0, 0)),
            pl.BlockSpec((1, N, N), lambda g: (g, 0, 0)),
            full(F, H), full(1, H),
            full(H, H), full(1, H),
            full(H, H), full(1, H),
            full(H, H), full(1, H),
            full(B, 1), full(B, N),
        ],
        out_specs=full(B, H),
        out_shape=jax.ShapeDtypeStruct((B, H), jnp.float32),
    )(cdfg_xs, cdfg_as, W_in, biases[0], W0, biases[1], W1, biases[2],
      W2, biases[3], idx, mask_f)

    return out
